# baseline (speedup 1.0000x reference)
import functools

import jax
import jax.numpy as jnp
from jax import lax
from jax.experimental import pallas as pl
from jax.experimental.pallas import tpu as pltpu

B = 8
N0 = 2048
N1 = 512
N2 = 256
KNN = 32
EPS = 1e-5
HI = jax.lax.Precision.HIGHEST


def _dot(a, b):
    return lax.dot_general(a, b, (((1,), (0,)), ((), ())),
                           preferred_element_type=jnp.float32)


def _dot_hi(a, b):
    return lax.dot_general(a, b, (((1,), (0,)), ((), ())),
                           preferred_element_type=jnp.float32, precision=HI)


def _split3(tab):
    hi = tab.astype(jnp.bfloat16)
    r1 = tab - hi.astype(jnp.float32)
    lo = r1.astype(jnp.bfloat16)
    lo2 = (r1 - lo.astype(jnp.float32)).astype(jnp.bfloat16)
    return jnp.concatenate([hi, lo, lo2], axis=1)


def _gather3(oh, tab3, c):
    g = lax.dot_general(oh, tab3, (((1,), (0,)), ((), ())),
                        preferred_element_type=jnp.float32)
    return (g[:, :c] + g[:, c:2 * c]) + g[:, 2 * c:]


def _dot_tt(a, b, ca, cb):
    return lax.dot_general(a, b, (((ca,), (cb,)), ((), ())),
                           preferred_element_type=jnp.float32)


def _mlp_in_body(x_ref, w1t_ref, w2t_ref, out_ref):
    x = x_ref[...]
    h = _dot(x, w1t_ref[...])
    m = jnp.mean(h, axis=0, keepdims=True)
    v = jnp.mean(h * h, axis=0, keepdims=True) - m * m
    h = jax.nn.relu((h - m) / jnp.sqrt(v + EPS))
    z = _dot(h, w2t_ref[...])
    m2 = jnp.mean(z, axis=0, keepdims=True)
    v2 = jnp.mean(z * z, axis=0, keepdims=True) - m2 * m2
    out_ref[...] = jax.nn.relu((z - m2) / jnp.sqrt(v2 + EPS))


def _fps_body(p0_ref, p1_ref, p2_ref, c1_ref, g0_ref, g1_ref, g2_ref,
              c2_ref, q0_ref, q1_ref, q2_ref):
    def run(p0, p1, p2, npoint, n, cent_ref, plane_refs):
        iota = lax.broadcasted_iota(jnp.int32, (B, n), 1)
        iota_np = lax.broadcasted_iota(jnp.int32, (B, npoint), 1)

        def body(i, carry):
            dist, far, cent, o0, o1, o2 = carry
            hitf = (iota_np == i).astype(jnp.float32)
            khit = 1.0 - hitf
            cent = cent * khit + far.astype(jnp.float32) * hitf
            sel = iota == far
            c0 = jnp.sum(jnp.where(sel, p0, 0.0), axis=1, keepdims=True)
            c1 = jnp.sum(jnp.where(sel, p1, 0.0), axis=1, keepdims=True)
            c2 = jnp.sum(jnp.where(sel, p2, 0.0), axis=1, keepdims=True)
            o0 = o0 * khit + c0 * hitf
            o1 = o1 * khit + c1 * hitf
            o2 = o2 * khit + c2 * hitf
            d = (p0 - c0) ** 2 + (p1 - c1) ** 2 + (p2 - c2) ** 2
            dist = jnp.minimum(dist, d)
            mx = jnp.max(dist, axis=1, keepdims=True)
            far = jnp.min(jnp.where(dist == mx, iota, n), axis=1, keepdims=True)
            return dist, far, cent, o0, o1, o2

        zf = p0[:, :npoint] * 0.0
        init = (p0 * 0.0 + 1e10,
                (p0[:, :1] * 0.0).astype(jnp.int32), zf, zf, zf, zf)
        _, _, cent, o0, o1, o2 = lax.fori_loop(0, npoint, body, init,
                                               unroll=8)
        cent_ref[...] = cent.astype(jnp.int32)
        plane_refs[0][...] = o0
        plane_refs[1][...] = o1
        plane_refs[2][...] = o2

    run(p0_ref[...], p1_ref[...], p2_ref[...], N1, N0, c1_ref,
        (g0_ref, g1_ref, g2_ref))
    run(g0_ref[...], g1_ref[...], g2_ref[...], N2, N1, c2_ref,
        (q0_ref, q1_ref, q2_ref))


def _group_body(n, nsrc, cout, xp_refs, q_ref, idx_ref, tab_ref, w_ref,
                h_ref, s1_ref, s2_ref, *, table_fn):
    tab = table_fn(tab_ref)
    cin = tab.shape[1]
    tab3 = _split3(tab)
    idx = idx_ref[0]
    iota_row = lax.broadcasted_iota(jnp.int32, (1, nsrc), 1)
    oh = (idx == iota_row).astype(jnp.bfloat16)
    npnt = _gather3(oh, tab3, cin)

    q = q_ref[0]
    planes = [r[0] for r in xp_refs]
    d = ((q[:, 0:1] - planes[0]) ** 2 + (q[:, 1:2] - planes[1]) ** 2
         + (q[:, 2:3] - planes[2]) ** 2)

    def body(k, carry):
        d, s1, s2 = carry
        mn = jnp.min(d, axis=1, keepdims=True)
        am = jnp.min(jnp.where(d == mn, iota_row, nsrc), axis=1, keepdims=True)
        ohk = (am == iota_row).astype(jnp.bfloat16)
        gk = _gather3(ohk, tab3, cin)
        feat = jnp.concatenate([gk - npnt, npnt], axis=1)
        hk = _dot(feat, w_ref[...])
        h_ref[:, pl.ds(k, 1)] = hk.reshape(1, 1, n, cout)
        s1 = s1 + jnp.sum(hk, axis=0, keepdims=True)
        s2 = s2 + jnp.sum(hk * hk, axis=0, keepdims=True)
        d = jnp.where(am == iota_row, 3.0e38, d)
        return d, s1, s2

    z = q[:1, :1] * 0.0 + jnp.zeros((1, cout), jnp.float32)
    _, s1, s2 = lax.fori_loop(0, KNN, body, (d, z, z))
    s1_ref[...] = s1.reshape(1, 1, cout)
    s2_ref[...] = s2.reshape(1, 1, cout)


def _group1_body(x0, x1, x2, q, idx, pts, w, h, s1, s2):
    _group_body(N1, N0, 128, (x0, x1, x2), q, idx, pts, w, h, s1, s2,
                table_fn=lambda r: r[0])


def _group2_body(g0, g1, g2, q, idx, zmax, p1, p2, w, h, s1, s2):
    def table(zr):
        cnt = float(B * N1 * KNN)
        s = jnp.sum(p1[...], axis=0)
        m = s / cnt
        v = jnp.sum(p2[...], axis=0) / cnt - m * m
        return jax.nn.relu((zr[0] - m) / jnp.sqrt(v + EPS))
    _group_body(N2, N1, 256, (g0, g1, g2), q, idx, zmax, w, h, s1, s2,
                table_fn=table)


def _conv2_body(h_ref, s1_ref, s2_ref, w_ref, zmax_ref, p1_ref, p2_ref, *,
                cnt):
    m = jnp.sum(s1_ref[...], axis=0) / cnt
    v = jnp.sum(s2_ref[...], axis=0) / cnt - m * m
    den = jnp.sqrt(v + EPS)
    w = w_ref[...]
    zmax = None
    pa = None
    pb = None
    for j in range(KNN):
        hn = jax.nn.relu((h_ref[0, j] - m) / den)
        z = _dot(hn, w)
        zmax = z if zmax is None else jnp.maximum(zmax, z)
        zs = jnp.sum(z, axis=0, keepdims=True)
        zq = jnp.sum(z * z, axis=0, keepdims=True)
        pa = zs if pa is None else pa + zs
        pb = zq if pb is None else pb + zq
    zmax_ref[...] = zmax[None]
    p1_ref[...] = pa[None]
    p2_ref[...] = pb[None]


def _bn_all(t):
    bsz, n, c = t.shape
    t2 = t.reshape(bsz * n, c)
    m = jnp.mean(t2, axis=0, keepdims=True)
    v = jnp.mean(t2 * t2, axis=0, keepdims=True) - m * m
    return ((t2 - m) / jnp.sqrt(v + EPS)).reshape(bsz, n, c)


def _per_batch(fn, x):
    return jnp.concatenate([fn(x[b])[None] for b in range(B)], axis=0)


def _attn_body(z2_ref, q1_ref, q2_ref, pw1_ref, pw2_ref, *rest):
    sa = []
    for l in range(4):
        sa.append(rest[5 * l:5 * l + 5])
    xcat_ref = rest[20]
    cnt = float(B * N2 * KNN)
    s = jnp.sum(q1_ref[...], axis=0)
    m = s / cnt
    v = jnp.sum(q2_ref[...], axis=0) / cnt - m * m
    f1 = jax.nn.relu((z2_ref[...] - m) / jnp.sqrt(v + EPS))

    x = jax.nn.relu(_bn_all(_per_batch(lambda t: _dot(t, pw1_ref[...]), f1)))
    x = jax.nn.relu(_bn_all(_per_batch(lambda t: _dot(t, pw2_ref[...]), x)))

    outs = []
    for l in range(4):
        wq, wv, bv, wt, bt = sa[l]
        ts = []
        for b in range(B):
            xb = x[b]
            qr = _dot(xb, wq[...])
            vr = _dot(xb, wv[...]) + bv[...]
            e = _dot_tt(qr, qr, 1, 1)
            a = jax.nn.softmax(e, axis=-1)
            a = a / (1e-9 + jnp.sum(a, axis=0, keepdims=True))
            xr = _dot_tt(a, vr, 0, 0)
            ts.append((_dot(xb - xr, wt[...]) + bt[...])[None])
        t = jnp.concatenate(ts, axis=0)
        x = x + jax.nn.relu(_bn_all(t))
        outs.append(x)
    xcat_ref[...] = jnp.concatenate(outs + [f1], axis=2)


def _fuse_body(xcat_ref, wf_ref, ymax_ref, r1_ref, r2_ref):
    y = _dot(xcat_ref[0], wf_ref[...])
    ymax_ref[...] = jnp.max(y, axis=0, keepdims=True)[None]
    r1_ref[...] = jnp.sum(y, axis=0, keepdims=True)[None]
    r2_ref[...] = jnp.sum(y * y, axis=0, keepdims=True)[None]


def _final_body(ymax_ref, r1_ref, r2_ref, out_ref):
    cnt = float(B * N2)
    s = jnp.sum(r1_ref[...], axis=0)
    m = s / cnt
    v = jnp.sum(r2_ref[...], axis=0) / cnt - m * m
    y = (ymax_ref[...] - m[None]) / jnp.sqrt(v + EPS)[None]
    out_ref[...] = jnp.where(y > 0, y, 0.2 * y)


def _f32(shape):
    return jax.ShapeDtypeStruct(shape, jnp.float32)


def _i32(shape):
    return jax.ShapeDtypeStruct(shape, jnp.int32)


@jax.jit
def kernel(x, params):
    p = params
    xyz = x[..., :3]
    f0 = xyz[:, :, 0]
    f1_ = xyz[:, :, 1]
    f2 = xyz[:, :, 2]
    xp0 = f0.reshape(B, 1, N0)
    xp1 = f1_.reshape(B, 1, N0)
    xp2 = f2.reshape(B, 1, N0)

    pts = pl.pallas_call(
        _mlp_in_body,
        out_shape=_f32((B * N0, 64)),
    )(x.reshape(B * N0, 7), p['W1'].T, p['W2'].T)
    pts = pts.reshape(B, N0, 64)

    c1r, g0r, g1r, g2r, c2r, q0r, q1r, q2r = pl.pallas_call(
        _fps_body,
        out_shape=(_i32((B, N1)), _f32((B, N1)), _f32((B, N1)),
                   _f32((B, N1)), _i32((B, N2)), _f32((B, N2)),
                   _f32((B, N2)), _f32((B, N2))),
    )(f0, f1_, f2)
    c1 = c1r[:, :, None]
    c2 = c2r[:, :, None]
    nxs = jnp.stack([g0r, g1r, g2r], axis=2)
    nx2s = jnp.stack([q0r, q1r, q2r], axis=2)
    g0 = g0r.reshape(B, 1, N1)
    g1 = g1r.reshape(B, 1, N1)
    g2 = g2r.reshape(B, 1, N1)

    w1g = p['L0W1'].T
    grid3 = pl.GridSpec(
        grid=(B,),
        in_specs=[
            pl.BlockSpec((1, 1, N0), lambda b: (b, 0, 0)),
            pl.BlockSpec((1, 1, N0), lambda b: (b, 0, 0)),
            pl.BlockSpec((1, 1, N0), lambda b: (b, 0, 0)),
            pl.BlockSpec((1, N1, 3), lambda b: (b, 0, 0)),
            pl.BlockSpec((1, N1, 1), lambda b: (b, 0, 0)),
            pl.BlockSpec((1, N0, 64), lambda b: (b, 0, 0)),
            pl.BlockSpec((128, 128), lambda b: (0, 0)),
        ],
        out_specs=[
            pl.BlockSpec((1, KNN, N1, 128), lambda b: (b, 0, 0, 0)),
            pl.BlockSpec((1, 1, 128), lambda b: (b, 0, 0)),
            pl.BlockSpec((1, 1, 128), lambda b: (b, 0, 0)),
        ],
    )
    h1, s1a, s1b = pl.pallas_call(
        _group1_body, grid_spec=grid3,
        out_shape=(_f32((B, KNN, N1, 128)), _f32((B, 1, 128)),
                   _f32((B, 1, 128))),
    )(xp0, xp1, xp2, nxs, c1, pts, w1g)

    grid4 = pl.GridSpec(
        grid=(B,),
        in_specs=[
            pl.BlockSpec((1, KNN, N1, 128), lambda b: (b, 0, 0, 0)),
            pl.BlockSpec((B, 1, 128), lambda b: (0, 0, 0)),
            pl.BlockSpec((B, 1, 128), lambda b: (0, 0, 0)),
            pl.BlockSpec((128, 128), lambda b: (0, 0)),
        ],
        out_specs=[
            pl.BlockSpec((1, N1, 128), lambda b: (b, 0, 0)),
            pl.BlockSpec((1, 1, 128), lambda b: (b, 0, 0)),
            pl.BlockSpec((1, 1, 128), lambda b: (b, 0, 0)),
        ],
    )
    zmax, p1a, p1b = pl.pallas_call(
        functools.partial(_conv2_body, cnt=float(B * N1 * KNN)),
        grid_spec=grid4,
        out_shape=(_f32((B, N1, 128)), _f32((B, 1, 128)), _f32((B, 1, 128))),
    )(h1, s1a, s1b, p['L0W2'].T)

    w2g = p['L1W1'].T
    grid6 = pl.GridSpec(
        grid=(B,),
        in_specs=[
            pl.BlockSpec((1, 1, N1), lambda b: (b, 0, 0)),
            pl.BlockSpec((1, 1, N1), lambda b: (b, 0, 0)),
            pl.BlockSpec((1, 1, N1), lambda b: (b, 0, 0)),
            pl.BlockSpec((1, N2, 3), lambda b: (b, 0, 0)),
            pl.BlockSpec((1, N2, 1), lambda b: (b, 0, 0)),
            pl.BlockSpec((1, N1, 128), lambda b: (b, 0, 0)),
            pl.BlockSpec((B, 1, 128), lambda b: (0, 0, 0)),
            pl.BlockSpec((B, 1, 128), lambda b: (0, 0, 0)),
            pl.BlockSpec((256, 256), lambda b: (0, 0)),
        ],
        out_specs=[
            pl.BlockSpec((1, KNN, N2, 256), lambda b: (b, 0, 0, 0)),
            pl.BlockSpec((1, 1, 256), lambda b: (b, 0, 0)),
            pl.BlockSpec((1, 1, 256), lambda b: (b, 0, 0)),
        ],
    )
    h2, s2a, s2b = pl.pallas_call(
        _group2_body, grid_spec=grid6,
        out_shape=(_f32((B, KNN, N2, 256)), _f32((B, 1, 256)),
                   _f32((B, 1, 256))),
    )(g0, g1, g2, nx2s, c2, zmax, p1a, p1b, w2g)

    grid7 = pl.GridSpec(
        grid=(B,),
        in_specs=[
            pl.BlockSpec((1, KNN, N2, 256), lambda b: (b, 0, 0, 0)),
            pl.BlockSpec((B, 1, 256), lambda b: (0, 0, 0)),
            pl.BlockSpec((B, 1, 256), lambda b: (0, 0, 0)),
            pl.BlockSpec((256, 256), lambda b: (0, 0)),
        ],
        out_specs=[
            pl.BlockSpec((1, N2, 256), lambda b: (b, 0, 0)),
            pl.BlockSpec((1, 1, 256), lambda b: (b, 0, 0)),
            pl.BlockSpec((1, 1, 256), lambda b: (b, 0, 0)),
        ],
    )
    z2max, p2a, p2b = pl.pallas_call(
        functools.partial(_conv2_body, cnt=float(B * N2 * KNN)),
        grid_spec=grid7,
        out_shape=(_f32((B, N2, 256)), _f32((B, 1, 256)), _f32((B, 1, 256))),
    )(h2, s2a, s2b, p['L1W2'].T)

    pt = p['pt']
    sa_args = []
    for name in ('sa1', 'sa2', 'sa3', 'sa4'):
        s = pt[name]
        sa_args += [s['Wq'].T, s['Wv'].T, s['bv'].reshape(1, 256),
                    s['Wt'].T, s['bt'].reshape(1, 256)]
    xcat = pl.pallas_call(
        _attn_body,
        out_shape=_f32((B, N2, 1280)),
    )(z2max, p2a, p2b, pt['W1'].T, pt['W2'].T, *sa_args)

    grid9 = pl.GridSpec(
        grid=(B,),
        in_specs=[
            pl.BlockSpec((1, N2, 1280), lambda b: (b, 0, 0)),
            pl.BlockSpec((1280, 1024), lambda b: (0, 0)),
        ],
        out_specs=[
            pl.BlockSpec((1, 1, 1024), lambda b: (b, 0, 0)),
            pl.BlockSpec((1, 1, 1024), lambda b: (b, 0, 0)),
            pl.BlockSpec((1, 1, 1024), lambda b: (b, 0, 0)),
        ],
    )
    ymax, r1, r2 = pl.pallas_call(
        _fuse_body, grid_spec=grid9,
        out_shape=(_f32((B, 1, 1024)), _f32((B, 1, 1024)), _f32((B, 1, 1024))),
    )(xcat, p['Wfuse'].T)

    out = pl.pallas_call(
        _final_body,
        out_shape=_f32((B, 1, 1024)),
    )(ymax, r1, r2)
    return out.reshape(B, 1024)


# speedup vs baseline: 1.1503x; 1.1503x over previous
import functools

import jax
import jax.numpy as jnp
from jax import lax
from jax.experimental import pallas as pl
from jax.experimental.pallas import tpu as pltpu
from jax.experimental.pallas import tpu_sc as plsc

SC_NC = 2
SC_NS = 16
SC_NW = SC_NC * SC_NS

B = 8
N0 = 2048
N1 = 512
N2 = 256
KNN = 32
EPS = 1e-5
HI = jax.lax.Precision.HIGHEST


def _dot(a, b):
    return lax.dot_general(a, b, (((1,), (0,)), ((), ())),
                           preferred_element_type=jnp.float32)


def _dot_hi(a, b):
    return lax.dot_general(a, b, (((1,), (0,)), ((), ())),
                           preferred_element_type=jnp.float32, precision=HI)


def _split3(tab):
    hi = tab.astype(jnp.bfloat16)
    r1 = tab - hi.astype(jnp.float32)
    lo = r1.astype(jnp.bfloat16)
    lo2 = (r1 - lo.astype(jnp.float32)).astype(jnp.bfloat16)
    return jnp.concatenate([hi, lo, lo2], axis=1)


def _gather3(oh, tab3, c):
    g = lax.dot_general(oh, tab3, (((1,), (0,)), ((), ())),
                        preferred_element_type=jnp.float32)
    return (g[:, :c] + g[:, c:2 * c]) + g[:, 2 * c:]


def _dot_tt(a, b, ca, cb):
    return lax.dot_general(a, b, (((ca,), (cb,)), ((), ())),
                           preferred_element_type=jnp.float32)


def _make_sc_gather(D, rpw):
    ch = 128
    nch = rpw // ch
    ngrp = nch // 4
    rem = nch - ngrp * 4
    mesh = plsc.VectorSubcoreMesh(core_axis_name="c", subcore_axis_name="s")

    def body(table_hbm, idx_hbm, out_hbm, idx_v, rows_v, sem):
        wid = lax.axis_index("s") * SC_NC + lax.axis_index("c")
        base = wid * rpw
        pltpu.sync_copy(idx_hbm.at[pl.ds(base, rpw)], idx_v)

        def grp(g, _):
            for j in range(4):
                off = g * (ch * 4) + j * ch
                pltpu.async_copy(table_hbm.at[idx_v.at[pl.ds(off, ch)]],
                                 rows_v.at[pl.ds(j * ch, ch)], sem)
            for j in range(4):
                pltpu.make_async_copy(table_hbm.at[idx_v.at[pl.ds(0, ch)]],
                                      rows_v.at[pl.ds(0, ch)], sem).wait()
            pltpu.sync_copy(rows_v,
                            out_hbm.at[pl.ds(base + g * (ch * 4), ch * 4)])
            return 0

        lax.fori_loop(0, ngrp, grp, 0, unroll=False)
        for j in range(rem):
            off = ngrp * (ch * 4) + j * ch
            pltpu.async_copy(table_hbm.at[idx_v.at[pl.ds(off, ch)]],
                             rows_v.at[pl.ds(j * ch, ch)], sem)
        for j in range(rem):
            pltpu.make_async_copy(table_hbm.at[idx_v.at[pl.ds(0, ch)]],
                                  rows_v.at[pl.ds(0, ch)], sem).wait()
        if rem:
            pltpu.sync_copy(rows_v.at[pl.ds(0, rem * ch)],
                            out_hbm.at[pl.ds(base + ngrp * (ch * 4),
                                             rem * ch)])

    import functools as _ft
    return _ft.partial(
        pl.kernel, body, mesh=mesh,
        out_type=jax.ShapeDtypeStruct((rpw * SC_NW, D), jnp.float32),
        scratch_types=[pltpu.VMEM((rpw,), jnp.int32),
                       pltpu.VMEM((ch * 4, D), jnp.float32),
                       pltpu.SemaphoreType.DMA])


def _sel1_body(x0_ref, x1_ref, x2_ref, q_ref, idx_ref):
    q = q_ref[0]
    planes = [x0_ref[0], x1_ref[0], x2_ref[0]]
    iota_row = lax.broadcasted_iota(jnp.int32, (1, N0), 1)
    lane_k = lax.broadcasted_iota(jnp.int32, (N1, KNN), 1)
    d = ((q[:, 0:1] - planes[0]) ** 2 + (q[:, 1:2] - planes[1]) ** 2
         + (q[:, 2:3] - planes[2]) ** 2)
    acc = d[:, :KNN] * 0.0
    for k in range(KNN):
        mn = jnp.min(d, axis=1, keepdims=True)
        am = jnp.min(jnp.where(d == mn, iota_row, N0), axis=1, keepdims=True)
        hit = (lane_k == k).astype(jnp.float32)
        acc = acc * (1.0 - hit) + am.astype(jnp.float32) * hit
        d = jnp.where(am == iota_row, 3.0e38, d)
    idx_ref[...] = acc.astype(jnp.int32)[None]


def _gconv1_body(g_ref, np_ref, w_ref, h_ref, s1_ref, s2_ref):
    np64 = np_ref[0][:, :64]
    w = w_ref[...]
    s1 = None
    s2 = None
    for k in range(KNN):
        gk = g_ref[k][:, :64]
        feat = jnp.concatenate([gk - np64, np64], axis=1)
        hk = _dot(feat, w)
        h_ref[0, k] = hk
        zs = jnp.sum(hk, axis=0, keepdims=True)
        zq = jnp.sum(hk * hk, axis=0, keepdims=True)
        s1 = zs if s1 is None else s1 + zs
        s2 = zq if s2 is None else s2 + zq
    s1_ref[...] = s1[None]
    s2_ref[...] = s2[None]


def _mlp_in_body(x_ref, w1t_ref, w2t_ref, out_ref):
    x = x_ref[...]
    h = _dot(x, w1t_ref[...])
    m = jnp.mean(h, axis=0, keepdims=True)
    v = jnp.mean(h * h, axis=0, keepdims=True) - m * m
    h = jax.nn.relu((h - m) / jnp.sqrt(v + EPS))
    z = _dot(h, w2t_ref[...])
    m2 = jnp.mean(z, axis=0, keepdims=True)
    v2 = jnp.mean(z * z, axis=0, keepdims=True) - m2 * m2
    r = jax.nn.relu((z - m2) / jnp.sqrt(v2 + EPS))
    out_ref[...] = jnp.concatenate([r, r * 0.0], axis=1)


def _fps_body(p0_ref, p1_ref, p2_ref, c1_ref, g0_ref, g1_ref, g2_ref,
              c2_ref, q0_ref, q1_ref, q2_ref):
    def run(p0, p1, p2, npoint, n, cent_ref, plane_refs):
        iota = lax.broadcasted_iota(jnp.int32, (B, n), 1)
        iota_np = lax.broadcasted_iota(jnp.int32, (B, npoint), 1)

        def body(i, carry):
            dist, far, cent, o0, o1, o2 = carry
            hitf = (iota_np == i).astype(jnp.float32)
            khit = 1.0 - hitf
            cent = cent * khit + far.astype(jnp.float32) * hitf
            sel = iota == far
            c0 = jnp.sum(jnp.where(sel, p0, 0.0), axis=1, keepdims=True)
            c1 = jnp.sum(jnp.where(sel, p1, 0.0), axis=1, keepdims=True)
            c2 = jnp.sum(jnp.where(sel, p2, 0.0), axis=1, keepdims=True)
            o0 = o0 * khit + c0 * hitf
            o1 = o1 * khit + c1 * hitf
            o2 = o2 * khit + c2 * hitf
            d = (p0 - c0) ** 2 + (p1 - c1) ** 2 + (p2 - c2) ** 2
            dist = jnp.minimum(dist, d)
            mx = jnp.max(dist, axis=1, keepdims=True)
            far = jnp.min(jnp.where(dist == mx, iota, n), axis=1, keepdims=True)
            return dist, far, cent, o0, o1, o2

        zf = p0[:, :npoint] * 0.0
        init = (p0 * 0.0 + 1e10,
                (p0[:, :1] * 0.0).astype(jnp.int32), zf, zf, zf, zf)
        _, _, cent, o0, o1, o2 = lax.fori_loop(0, npoint, body, init,
                                               unroll=8)
        cent_ref[...] = cent.astype(jnp.int32)
        plane_refs[0][...] = o0
        plane_refs[1][...] = o1
        plane_refs[2][...] = o2

    run(p0_ref[...], p1_ref[...], p2_ref[...], N1, N0, c1_ref,
        (g0_ref, g1_ref, g2_ref))
    run(g0_ref[...], g1_ref[...], g2_ref[...], N2, N1, c2_ref,
        (q0_ref, q1_ref, q2_ref))


def _group_body(n, nsrc, cout, xp_refs, q_ref, idx_ref, tab_ref, w_ref,
                h_ref, s1_ref, s2_ref, *, table_fn):
    tab = table_fn(tab_ref)
    cin = tab.shape[1]
    tab3 = _split3(tab)
    idx = idx_ref[0]
    iota_row = lax.broadcasted_iota(jnp.int32, (1, nsrc), 1)
    oh = (idx == iota_row).astype(jnp.bfloat16)
    npnt = _gather3(oh, tab3, cin)

    q = q_ref[0]
    planes = [r[0] for r in xp_refs]
    d = ((q[:, 0:1] - planes[0]) ** 2 + (q[:, 1:2] - planes[1]) ** 2
         + (q[:, 2:3] - planes[2]) ** 2)

    def body(k, carry):
        d, s1, s2 = carry
        mn = jnp.min(d, axis=1, keepdims=True)
        am = jnp.min(jnp.where(d == mn, iota_row, nsrc), axis=1, keepdims=True)
        ohk = (am == iota_row).astype(jnp.bfloat16)
        gk = _gather3(ohk, tab3, cin)
        feat = jnp.concatenate([gk - npnt, npnt], axis=1)
        hk = _dot(feat, w_ref[...])
        h_ref[:, pl.ds(k, 1)] = hk.reshape(1, 1, n, cout)
        s1 = s1 + jnp.sum(hk, axis=0, keepdims=True)
        s2 = s2 + jnp.sum(hk * hk, axis=0, keepdims=True)
        d = jnp.where(am == iota_row, 3.0e38, d)
        return d, s1, s2

    z = q[:1, :1] * 0.0 + jnp.zeros((1, cout), jnp.float32)
    _, s1, s2 = lax.fori_loop(0, KNN, body, (d, z, z))
    s1_ref[...] = s1.reshape(1, 1, cout)
    s2_ref[...] = s2.reshape(1, 1, cout)


def _group1_body(x0, x1, x2, q, idx, pts, w, h, s1, s2):
    _group_body(N1, N0, 128, (x0, x1, x2), q, idx, pts, w, h, s1, s2,
                table_fn=lambda r: r[0])


def _group2_body(g0, g1, g2, q, idx, zmax, p1, p2, w, h, s1, s2):
    def table(zr):
        cnt = float(B * N1 * KNN)
        s = jnp.sum(p1[...], axis=0)
        m = s / cnt
        v = jnp.sum(p2[...], axis=0) / cnt - m * m
        return jax.nn.relu((zr[0] - m) / jnp.sqrt(v + EPS))
    _group_body(N2, N1, 256, (g0, g1, g2), q, idx, zmax, w, h, s1, s2,
                table_fn=table)


def _conv2_body(h_ref, s1_ref, s2_ref, w_ref, zmax_ref, p1_ref, p2_ref, *,
                cnt):
    m = jnp.sum(s1_ref[...], axis=0) / cnt
    v = jnp.sum(s2_ref[...], axis=0) / cnt - m * m
    den = jnp.sqrt(v + EPS)
    w = w_ref[...]
    zmax = None
    pa = None
    pb = None
    for j in range(KNN):
        hn = jax.nn.relu((h_ref[0, j] - m) / den)
        z = _dot(hn, w)
        zmax = z if zmax is None else jnp.maximum(zmax, z)
        zs = jnp.sum(z, axis=0, keepdims=True)
        zq = jnp.sum(z * z, axis=0, keepdims=True)
        pa = zs if pa is None else pa + zs
        pb = zq if pb is None else pb + zq
    zmax_ref[...] = zmax[None]
    p1_ref[...] = pa[None]
    p2_ref[...] = pb[None]


def _bn_all(t):
    bsz, n, c = t.shape
    t2 = t.reshape(bsz * n, c)
    m = jnp.mean(t2, axis=0, keepdims=True)
    v = jnp.mean(t2 * t2, axis=0, keepdims=True) - m * m
    return ((t2 - m) / jnp.sqrt(v + EPS)).reshape(bsz, n, c)


def _per_batch(fn, x):
    return jnp.concatenate([fn(x[b])[None] for b in range(B)], axis=0)


def _attn_body(z2_ref, q1_ref, q2_ref, pw1_ref, pw2_ref, *rest):
    sa = []
    for l in range(4):
        sa.append(rest[5 * l:5 * l + 5])
    xcat_ref = rest[20]
    cnt = float(B * N2 * KNN)
    s = jnp.sum(q1_ref[...], axis=0)
    m = s / cnt
    v = jnp.sum(q2_ref[...], axis=0) / cnt - m * m
    f1 = jax.nn.relu((z2_ref[...] - m) / jnp.sqrt(v + EPS))

    x = jax.nn.relu(_bn_all(_per_batch(lambda t: _dot(t, pw1_ref[...]), f1)))
    x = jax.nn.relu(_bn_all(_per_batch(lambda t: _dot(t, pw2_ref[...]), x)))

    outs = []
    for l in range(4):
        wq, wv, bv, wt, bt = sa[l]
        ts = []
        for b in range(B):
            xb = x[b]
            qr = _dot(xb, wq[...])
            vr = _dot(xb, wv[...]) + bv[...]
            e = _dot_tt(qr, qr, 1, 1)
            a = jax.nn.softmax(e, axis=-1)
            a = a / (1e-9 + jnp.sum(a, axis=0, keepdims=True))
            xr = _dot_tt(a, vr, 0, 0)
            ts.append((_dot(xb - xr, wt[...]) + bt[...])[None])
        t = jnp.concatenate(ts, axis=0)
        x = x + jax.nn.relu(_bn_all(t))
        outs.append(x)
    xcat_ref[...] = jnp.concatenate(outs + [f1], axis=2)


def _fuse_body(xcat_ref, wf_ref, ymax_ref, r1_ref, r2_ref):
    y = _dot(xcat_ref[0], wf_ref[...])
    ymax_ref[...] = jnp.max(y, axis=0, keepdims=True)[None]
    r1_ref[...] = jnp.sum(y, axis=0, keepdims=True)[None]
    r2_ref[...] = jnp.sum(y * y, axis=0, keepdims=True)[None]


def _final_body(ymax_ref, r1_ref, r2_ref, out_ref):
    cnt = float(B * N2)
    s = jnp.sum(r1_ref[...], axis=0)
    m = s / cnt
    v = jnp.sum(r2_ref[...], axis=0) / cnt - m * m
    y = (ymax_ref[...] - m[None]) / jnp.sqrt(v + EPS)[None]
    out_ref[...] = jnp.where(y > 0, y, 0.2 * y)


def _f32(shape):
    return jax.ShapeDtypeStruct(shape, jnp.float32)


def _i32(shape):
    return jax.ShapeDtypeStruct(shape, jnp.int32)


@jax.jit
def kernel(x, params):
    p = params
    xyz = x[..., :3]
    f0 = xyz[:, :, 0]
    f1_ = xyz[:, :, 1]
    f2 = xyz[:, :, 2]
    xp0 = f0.reshape(B, 1, N0)
    xp1 = f1_.reshape(B, 1, N0)
    xp2 = f2.reshape(B, 1, N0)

    pts = pl.pallas_call(
        _mlp_in_body,
        out_shape=_f32((B * N0, 128)),
    )(x.reshape(B * N0, 7), p['W1'].T, p['W2'].T)

    c1r, g0r, g1r, g2r, c2r, q0r, q1r, q2r = pl.pallas_call(
        _fps_body,
        out_shape=(_i32((B, N1)), _f32((B, N1)), _f32((B, N1)),
                   _f32((B, N1)), _i32((B, N2)), _f32((B, N2)),
                   _f32((B, N2)), _f32((B, N2))),
    )(f0, f1_, f2)
    c1 = c1r[:, :, None]
    c2 = c2r[:, :, None]
    nxs = jnp.stack([g0r, g1r, g2r], axis=2)
    nx2s = jnp.stack([q0r, q1r, q2r], axis=2)
    g0 = g0r.reshape(B, 1, N1)
    g1 = g1r.reshape(B, 1, N1)
    g2 = g2r.reshape(B, 1, N1)

    grid3a = pl.GridSpec(
        grid=(B,),
        in_specs=[
            pl.BlockSpec((1, 1, N0), lambda b: (b, 0, 0)),
            pl.BlockSpec((1, 1, N0), lambda b: (b, 0, 0)),
            pl.BlockSpec((1, 1, N0), lambda b: (b, 0, 0)),
            pl.BlockSpec((1, N1, 3), lambda b: (b, 0, 0)),
        ],
        out_specs=pl.BlockSpec((1, N1, KNN), lambda b: (b, 0, 0)),
    )
    idx1 = pl.pallas_call(
        _sel1_body, grid_spec=grid3a,
        out_shape=_i32((B, N1, KNN)),
    )(xp0, xp1, xp2, nxs)

    boff = (jnp.arange(B, dtype=jnp.int32) * N0)
    gidx = (jnp.transpose(idx1, (0, 2, 1))
            + boff[:, None, None]).reshape(-1)
    npidx = (c1r + boff[:, None]).reshape(-1)
    idx_all = jnp.concatenate([gidx, npidx])
    rpw1 = idx_all.shape[0] // SC_NW
    gat = _make_sc_gather(128, rpw1)()(pts, idx_all)
    g3 = gat.reshape(-1, N1, 128)

    w1g = p['L0W1'].T
    grid3c = pl.GridSpec(
        grid=(B,),
        in_specs=[
            pl.BlockSpec((KNN, N1, 128), lambda b: (b, 0, 0)),
            pl.BlockSpec((1, N1, 128), lambda b: (B * KNN + b, 0, 0)),
            pl.BlockSpec((128, 128), lambda b: (0, 0)),
        ],
        out_specs=[
            pl.BlockSpec((1, KNN, N1, 128), lambda b: (b, 0, 0, 0)),
            pl.BlockSpec((1, 1, 128), lambda b: (b, 0, 0)),
            pl.BlockSpec((1, 1, 128), lambda b: (b, 0, 0)),
        ],
    )
    h1, s1a, s1b = pl.pallas_call(
        _gconv1_body, grid_spec=grid3c,
        out_shape=(_f32((B, KNN, N1, 128)), _f32((B, 1, 128)),
                   _f32((B, 1, 128))),
    )(g3, g3, w1g)

    grid4 = pl.GridSpec(
        grid=(B,),
        in_specs=[
            pl.BlockSpec((1, KNN, N1, 128), lambda b: (b, 0, 0, 0)),
            pl.BlockSpec((B, 1, 128), lambda b: (0, 0, 0)),
            pl.BlockSpec((B, 1, 128), lambda b: (0, 0, 0)),
            pl.BlockSpec((128, 128), lambda b: (0, 0)),
        ],
        out_specs=[
            pl.BlockSpec((1, N1, 128), lambda b: (b, 0, 0)),
            pl.BlockSpec((1, 1, 128), lambda b: (b, 0, 0)),
            pl.BlockSpec((1, 1, 128), lambda b: (b, 0, 0)),
        ],
    )
    zmax, p1a, p1b = pl.pallas_call(
        functools.partial(_conv2_body, cnt=float(B * N1 * KNN)),
        grid_spec=grid4,
        out_shape=(_f32((B, N1, 128)), _f32((B, 1, 128)), _f32((B, 1, 128))),
    )(h1, s1a, s1b, p['L0W2'].T)

    w2g = p['L1W1'].T
    grid6 = pl.GridSpec(
        grid=(B,),
        in_specs=[
            pl.BlockSpec((1, 1, N1), lambda b: (b, 0, 0)),
            pl.BlockSpec((1, 1, N1), lambda b: (b, 0, 0)),
            pl.BlockSpec((1, 1, N1), lambda b: (b, 0, 0)),
            pl.BlockSpec((1, N2, 3), lambda b: (b, 0, 0)),
            pl.BlockSpec((1, N2, 1), lambda b: (b, 0, 0)),
            pl.BlockSpec((1, N1, 128), lambda b: (b, 0, 0)),
            pl.BlockSpec((B, 1, 128), lambda b: (0, 0, 0)),
            pl.BlockSpec((B, 1, 128), lambda b: (0, 0, 0)),
            pl.BlockSpec((256, 256), lambda b: (0, 0)),
        ],
        out_specs=[
            pl.BlockSpec((1, KNN, N2, 256), lambda b: (b, 0, 0, 0)),
            pl.BlockSpec((1, 1, 256), lambda b: (b, 0, 0)),
            pl.BlockSpec((1, 1, 256), lambda b: (b, 0, 0)),
        ],
    )
    h2, s2a, s2b = pl.pallas_call(
        _group2_body, grid_spec=grid6,
        out_shape=(_f32((B, KNN, N2, 256)), _f32((B, 1, 256)),
                   _f32((B, 1, 256))),
    )(g0, g1, g2, nx2s, c2, zmax, p1a, p1b, w2g)

    grid7 = pl.GridSpec(
        grid=(B,),
        in_specs=[
            pl.BlockSpec((1, KNN, N2, 256), lambda b: (b, 0, 0, 0)),
            pl.BlockSpec((B, 1, 256), lambda b: (0, 0, 0)),
            pl.BlockSpec((B, 1, 256), lambda b: (0, 0, 0)),
            pl.BlockSpec((256, 256), lambda b: (0, 0)),
        ],
        out_specs=[
            pl.BlockSpec((1, N2, 256), lambda b: (b, 0, 0)),
            pl.BlockSpec((1, 1, 256), lambda b: (b, 0, 0)),
            pl.BlockSpec((1, 1, 256), lambda b: (b, 0, 0)),
        ],
    )
    z2max, p2a, p2b = pl.pallas_call(
        functools.partial(_conv2_body, cnt=float(B * N2 * KNN)),
        grid_spec=grid7,
        out_shape=(_f32((B, N2, 256)), _f32((B, 1, 256)), _f32((B, 1, 256))),
    )(h2, s2a, s2b, p['L1W2'].T)

    pt = p['pt']
    sa_args = []
    for name in ('sa1', 'sa2', 'sa3', 'sa4'):
        s = pt[name]
        sa_args += [s['Wq'].T, s['Wv'].T, s['bv'].reshape(1, 256),
                    s['Wt'].T, s['bt'].reshape(1, 256)]
    xcat = pl.pallas_call(
        _attn_body,
        out_shape=_f32((B, N2, 1280)),
    )(z2max, p2a, p2b, pt['W1'].T, pt['W2'].T, *sa_args)

    grid9 = pl.GridSpec(
        grid=(B,),
        in_specs=[
            pl.BlockSpec((1, N2, 1280), lambda b: (b, 0, 0)),
            pl.BlockSpec((1280, 1024), lambda b: (0, 0)),
        ],
        out_specs=[
            pl.BlockSpec((1, 1, 1024), lambda b: (b, 0, 0)),
            pl.BlockSpec((1, 1, 1024), lambda b: (b, 0, 0)),
            pl.BlockSpec((1, 1, 1024), lambda b: (b, 0, 0)),
        ],
    )
    ymax, r1, r2 = pl.pallas_call(
        _fuse_body, grid_spec=grid9,
        out_shape=(_f32((B, 1, 1024)), _f32((B, 1, 1024)), _f32((B, 1, 1024))),
    )(xcat, p['Wfuse'].T)

    out = pl.pallas_call(
        _final_body,
        out_shape=_f32((B, 1, 1024)),
    )(ymax, r1, r2)
    return out.reshape(B, 1024)
